# baseline (speedup 1.0000x reference)
import functools

import jax
import jax.numpy as jnp
from jax import lax
from jax.experimental import pallas as pl
from jax.experimental.pallas import tpu as pltpu
from jax.experimental.pallas import tpu_sc as plsc

NUM_GRAPHS = 64
EPS = 1e-5

_MESH = plsc.VectorSubcoreMesh(core_axis_name="c", subcore_axis_name="s")
_NOTILE = pltpu.CompilerParams(use_tc_tiling_on_sc=False)
_NW = 32


def _sc_gather(table, idx2, chr_):
    V, D = table.shape
    R = idx2.shape[0]
    N = R * 128
    che = chr_ * 128
    nch = R // chr_
    npw = -(-nch // _NW)

    @functools.partial(
        pl.kernel, mesh=_MESH,
        out_type=jax.ShapeDtypeStruct((N, D), jnp.float32),
        scratch_types=[
            pltpu.VMEM((chr_, 128), jnp.int32),
            pltpu.VMEM((che, D), jnp.float32),
            pltpu.SemaphoreType.DMA,
        ],
        compiler_params=_NOTILE,
    )
    def k(tab, idx, out, idx_v, buf_v, sem):
        wid = lax.axis_index("s") * 2 + lax.axis_index("c")

        def body(i, carry):
            ch = i * _NW + wid

            @pl.when(ch < nch)
            def _():
                pltpu.sync_copy(idx.at[pl.ds(ch * chr_, chr_)], idx_v)
                cps = [pltpu.async_copy(tab.at[idx_v.at[j]],
                                        buf_v.at[pl.ds(j * 128, 128)], sem)
                       for j in range(chr_)]
                for c in cps:
                    c.wait()
                pltpu.sync_copy(buf_v, out.at[pl.ds(ch * che, che)])
            return carry

        lax.fori_loop(0, npw, body, 0)

    return k(table, idx2)


def _sc_scatter16(msgs, idx2, n_seg):
    N, D = msgs.shape
    R = idx2.shape[0]
    chr_ = 10
    che = chr_ * 128
    nch = R // chr_
    npw = -(-nch // 16)
    ck = n_seg // 8
    acc_rows = ck + 960
    dump = ck + 500
    fp = 1000
    nfp = ck // fp

    @functools.partial(
        pl.kernel, mesh=_MESH,
        out_type=jax.ShapeDtypeStruct((n_seg, D), jnp.float32),
        scratch_types=[
            pltpu.VMEM((chr_, 128), jnp.int32),
            pltpu.VMEM((chr_, 128), jnp.int32),
            pltpu.VMEM((che, D), jnp.float32),
            pltpu.VMEM((128, D), jnp.float32),
            pltpu.VMEM_SHARED((acc_rows, D), jnp.float32),
            pltpu.SemaphoreType.DMA,
        ],
        compiler_params=_NOTILE,
    )
    def k(msg, idx, out, idx_v, idxp_v, buf_v, z_v, acc, sem):
        cid = lax.axis_index("c")
        sid = lax.axis_index("s")

        def zr(i, c):
            z_v[i] = jnp.zeros((D,), jnp.float32)
            return c
        lax.fori_loop(0, 128, zr, 0)
        for p in range(4):
            lo = (cid * 4 + p) * ck
            def zs(i, c):
                acc_r = (sid * (acc_rows // 16 // 128) + i) * 128
                pltpu.sync_copy(z_v, acc.at[pl.ds(acc_r, 128)])
                return c
            lax.fori_loop(0, acc_rows // 16 // 128, zs, 0)
            plsc.subcore_barrier()

            def sweep(i, c):
                ch = i * 16 + sid

                @pl.when(ch < nch)
                def _():
                    pltpu.sync_copy(idx.at[pl.ds(ch * chr_, chr_)], idx_v)
                    pltpu.sync_copy(msg.at[pl.ds(ch * che, che)], buf_v)
                    for j in range(chr_):
                        for g in range(8):
                            iv = idx_v[j, pl.ds(g * 16, 16)]
                            ok = (iv >= lo) & (iv < lo + ck)
                            idxp_v[j, pl.ds(g * 16, 16)] = jnp.where(
                                ok, iv - lo, dump)
                    for j in range(chr_):
                        pltpu.sync_copy(buf_v.at[pl.ds(j * 128, 128)],
                                        acc.at[idxp_v.at[j]], add=True)
                return c
            lax.fori_loop(0, npw, sweep, 0)
            plsc.subcore_barrier()

            def fl(i, c):
                pi = i * 16 + sid

                @pl.when(pi < nfp)
                def _():
                    r = pi * fp
                    pltpu.sync_copy(acc.at[pl.ds(r, fp)],
                                    buf_v.at[pl.ds(0, fp)])
                    pltpu.sync_copy(buf_v.at[pl.ds(0, fp)],
                                    out.at[pl.ds(lo + r, fp)])
                return c
            lax.fori_loop(0, -(-nfp // 16), fl, 0)
            plsc.subcore_barrier()

    return k(msgs, idx2)


def _sc_scatter128(msgs, idx2, n_seg):
    N, D = msgs.shape
    R = idx2.shape[0]
    chr_ = 4
    che = chr_ * 128
    nch = R // chr_
    npw = -(-nch // _NW)
    ck = n_seg // 4
    acc_rows = ck + 444
    dump = ck + 300
    fp = 125
    nfp = ck // fp

    @functools.partial(
        pl.kernel, mesh=_MESH,
        out_type=jax.ShapeDtypeStruct((2, n_seg, D), jnp.float32),
        scratch_types=[
            pltpu.VMEM((chr_, 128), jnp.int32),
            pltpu.VMEM((chr_, 128), jnp.int32),
            pltpu.VMEM((che, D), jnp.float32),
            pltpu.VMEM((128, D), jnp.float32),
            pltpu.VMEM_SHARED((acc_rows, D), jnp.float32),
            pltpu.SemaphoreType.DMA,
        ],
        compiler_params=_NOTILE,
    )
    def k(msg, idx, out, idx_v, idxp_v, buf_v, z_v, acc, sem):
        cid = lax.axis_index("c")
        sid = lax.axis_index("s")

        def zr(i, c):
            z_v[i] = jnp.zeros((D,), jnp.float32)
            return c
        lax.fori_loop(0, 128, zr, 0)
        for p in range(4):
            lo = p * ck

            def zs(i, c):
                pi = i * 16 + sid

                @pl.when(pi < acc_rows // 128)
                def _():
                    pltpu.sync_copy(z_v, acc.at[pl.ds(pi * 128, 128)])
                return c
            lax.fori_loop(0, -(-(acc_rows // 128) // 16), zs, 0)
            plsc.subcore_barrier()

            def sweep(i, c):
                ch = i * _NW + sid * 2 + cid

                @pl.when(ch < nch)
                def _():
                    pltpu.sync_copy(idx.at[pl.ds(ch * chr_, chr_)], idx_v)
                    pltpu.sync_copy(msg.at[pl.ds(ch * che, che)], buf_v)
                    for j in range(chr_):
                        for g in range(8):
                            iv = idx_v[j, pl.ds(g * 16, 16)]
                            ok = (iv >= lo) & (iv < lo + ck)
                            idxp_v[j, pl.ds(g * 16, 16)] = jnp.where(
                                ok, iv - lo, dump)
                    for j in range(chr_):
                        pltpu.sync_copy(buf_v.at[pl.ds(j * 128, 128)],
                                        acc.at[idxp_v.at[j]], add=True)
                return c
            lax.fori_loop(0, npw, sweep, 0)
            plsc.subcore_barrier()

            def fl(i, c):
                pi = i * 16 + sid

                @pl.when(pi < nfp)
                def _():
                    r = pi * fp
                    pltpu.sync_copy(acc.at[pl.ds(r, fp)],
                                    buf_v.at[pl.ds(0, fp)])
                    pltpu.sync_copy(buf_v.at[pl.ds(0, fp)],
                                    out.at[cid].at[pl.ds(lo + r, fp)])
                return c
            lax.fori_loop(0, -(-nfp // 16), fl, 0)
            plsc.subcore_barrier()

    return k(msgs, idx2)


def _rowmm(xs, Ws, b, adds=(), res=None, relu=False, blk=4000):
    n = xs[0].shape[0]
    M = Ws[0].shape[1]
    Wcat = jnp.concatenate(Ws, axis=0)
    K = Wcat.shape[0]
    b8 = jnp.tile(b.reshape(1, M), (8, 1))
    n_x = len(xs)
    n_a = len(adds)
    has_res = res is not None
    grid = n // blk

    def body(*refs):
        xr = refs[:n_x]
        Wr = refs[n_x]
        br = refs[n_x + 1]
        ar = refs[n_x + 2:n_x + 2 + n_a]
        rr = refs[n_x + 2 + n_a] if has_res else None
        out = refs[-1]
        k0 = 0
        acc = None
        for x in xr:
            k = x.shape[1]
            p = jnp.dot(x[...], Wr[pl.ds(k0, k), :],
                        preferred_element_type=jnp.float32)
            acc = p if acc is None else acc + p
            k0 += k
        acc = acc + br[0:1, :]
        for a, col in zip(ar, add_cols):
            acc = acc + a[...][:, col * M:(col + 1) * M]
        if relu:
            acc = jnp.maximum(acc, 0.0)
        if has_res:
            acc = acc + rr[...]
        out[...] = acc

    in_specs = [pl.BlockSpec((blk, x.shape[1]), lambda i: (i, 0)) for x in xs]
    in_specs.append(pl.BlockSpec((K, M), lambda i: (0, 0)))
    in_specs.append(pl.BlockSpec((8, M), lambda i: (0, 0)))
    operands = list(xs) + [Wcat, b8]
    add_cols = []
    for a in adds:
        arr, col = a if isinstance(a, tuple) else (a, 0)
        in_specs.append(pl.BlockSpec((blk, arr.shape[1]), lambda i: (i, 0)))
        operands.append(arr)
        add_cols.append(col)
    if has_res:
        in_specs.append(pl.BlockSpec((blk, M), lambda i: (i, 0)))
        operands.append(res)
    return pl.pallas_call(
        body,
        grid=(grid,),
        in_specs=in_specs,
        out_specs=pl.BlockSpec((blk, M), lambda i: (i, 0)),
        out_shape=jax.ShapeDtypeStruct((n, M), jnp.float32),
    )(*operands)


def _graph_norm(v, batch, w, bias):
    F = v.shape[-1]
    cnt = jax.ops.segment_sum(jnp.ones((v.shape[0],), jnp.float32), batch,
                              num_segments=NUM_GRAPHS)
    denom = jnp.maximum(cnt * F, 1.0)
    mean = jax.ops.segment_sum(v.sum(-1), batch, num_segments=NUM_GRAPHS) / denom
    var = (jax.ops.segment_sum((v * v).sum(-1), batch, num_segments=NUM_GRAPHS)
           / denom - mean * mean)
    inv = 1.0 / jnp.sqrt(var + EPS)
    return (v - mean[batch][:, None]) * inv[batch][:, None] * w + bias


def kernel(x, edge_attr, angle_attr, params, node_batch, edge_index,
           edge_batch, threebody_index, angle_batch):
    D = x.shape[1]
    ED = edge_attr.shape[1]
    N_E = edge_attr.shape[0]
    N_T = angle_attr.shape[0]
    src_e2 = edge_index[0].reshape(N_E // 128, 128)
    dst_e2 = edge_index[1].reshape(N_E // 128, 128)
    src_t2 = threebody_index[0].reshape(N_T // 128, 128)
    dst_t2 = threebody_index[1].reshape(N_T // 128, 128)
    src_e, dst_e = edge_index[0], edge_index[1]
    dst_t = threebody_index[1]
    e = edge_attr
    a = angle_attr
    for lp in params['layers']:
        Wa, Wb, Wc = lp['ne_W'][:D], lp['ne_W'][D:2 * D], lp['ne_W'][2 * D:]
        A, B, C = lp['ea_W'][:ED], lp['ea_W'][ED:2 * ED], lp['ea_W'][2 * ED:]
        Wm1, Wm2 = lp['emp_Wm'][:ED], lp['emp_Wm'][ED:]
        Wu1, Wu2 = lp['emp_Wu'][:ED], lp['emp_Wu'][ED:]
        Nm1, Nm2 = lp['nmp_Wm'][:D], lp['nmp_Wm'][D:]
        Nu1, Nu2 = lp['nmp_Wu'][:D], lp['nmp_Wu'][D:]

        xab = _rowmm([x], [jnp.concatenate([Wa, Wb], axis=1)],
                     jnp.zeros((32,), jnp.float32), blk=1000)
        xm = _rowmm([x], [Nm1], jnp.zeros((128,), jnp.float32), blk=1000)

        ga = _sc_gather(xab[:, :ED], src_e2, 10)
        gb = _sc_gather(xab[:, ED:], dst_e2, 10)
        e_pre = _rowmm([e], [Wc], lp['ne_b'], adds=(ga, gb), res=e)
        e1 = _graph_norm(e_pre, edge_batch, lp['en_w'], lp['en_b'])

        epA = _rowmm([e1], [jnp.concatenate([A, Wm1], axis=1)],
                     jnp.zeros((32,), jnp.float32))
        epB = _rowmm([e1], [B], jnp.zeros((ED,), jnp.float32))

        gAM = _sc_gather(epA, src_t2, 10)
        gB = _sc_gather(epB, dst_t2, 10)

        a_pre = _rowmm([a], [C], lp['ea_b'], adds=((gAM, 0), gB), res=a)
        a1 = _graph_norm(a_pre, angle_batch, lp['an_w'], lp['an_b'])

        m = _rowmm([a1], [Wm2], lp['emp_bm'], adds=((gAM, 1),), relu=True)
        agg_e = _sc_scatter16(m, dst_t2, N_E)
        e2 = _rowmm([e1, agg_e], [Wu1, Wu2], lp['emp_bu'], relu=True, res=e1)

        g4 = _sc_gather(xm, src_e2, 4)
        m2 = _rowmm([e2], [Nm2], lp['nmp_bm'], adds=(g4,), relu=True)
        agg_n2 = _sc_scatter128(m2, dst_e2, x.shape[0])
        agg_n = agg_n2[0] + agg_n2[1]
        x = _rowmm([x, agg_n], [Nu1, Nu2], lp['nmp_bu'], relu=True, res=x,
                   blk=1000)
        e = e2
        a = a1
    return x


# speedup vs baseline: 1.6550x; 1.6550x over previous
import functools

import jax
import jax.numpy as jnp
from jax import lax
from jax.experimental import pallas as pl
from jax.experimental.pallas import tpu as pltpu
from jax.experimental.pallas import tpu_sc as plsc

NUM_GRAPHS = 64
EPS = 1e-5

_MESH = plsc.VectorSubcoreMesh(core_axis_name="c", subcore_axis_name="s")
_NOTILE = pltpu.CompilerParams(use_tc_tiling_on_sc=False)
_NW = 32


def _sc_gather(table, idx2, chr_):
    V, D = table.shape
    R = idx2.shape[0]
    N = R * 128
    che = chr_ * 128
    nch = R // chr_
    npw = -(-nch // _NW)

    @functools.partial(
        pl.kernel, mesh=_MESH,
        out_type=jax.ShapeDtypeStruct((N, D), jnp.float32),
        scratch_types=[
            pltpu.VMEM((chr_, 128), jnp.int32),
            pltpu.VMEM((che, D), jnp.float32),
            pltpu.SemaphoreType.DMA,
        ],
        compiler_params=_NOTILE,
    )
    def k(tab, idx, out, idx_v, buf_v, sem):
        wid = lax.axis_index("s") * 2 + lax.axis_index("c")

        def body(i, carry):
            ch = i * _NW + wid

            @pl.when(ch < nch)
            def _():
                pltpu.sync_copy(idx.at[pl.ds(ch * chr_, chr_)], idx_v)
                cps = [pltpu.async_copy(tab.at[idx_v.at[j]],
                                        buf_v.at[pl.ds(j * 128, 128)], sem)
                       for j in range(chr_)]
                for c in cps:
                    c.wait()
                pltpu.sync_copy(buf_v, out.at[pl.ds(ch * che, che)])
            return carry

        lax.fori_loop(0, npw, body, 0)

    return k(table, idx2)


def _sc_scatter16(msgs, idx2, n_seg):
    N, D = msgs.shape
    R = idx2.shape[0]
    chr_ = 10
    che = chr_ * 128
    nch = R // chr_
    npw = -(-nch // 16)
    ck = n_seg // 8
    acc_rows = ck + 960
    dump = ck + 500
    fp = 1000
    nfp = ck // fp

    @functools.partial(
        pl.kernel, mesh=_MESH,
        out_type=jax.ShapeDtypeStruct((n_seg, D), jnp.float32),
        scratch_types=[
            pltpu.VMEM((chr_, 128), jnp.int32),
            pltpu.VMEM((chr_, 128), jnp.int32),
            pltpu.VMEM((che, D), jnp.float32),
            pltpu.VMEM((128, D), jnp.float32),
            pltpu.VMEM_SHARED((acc_rows, D), jnp.float32),
            pltpu.SemaphoreType.DMA,
        ],
        compiler_params=_NOTILE,
    )
    def k(msg, idx, out, idx_v, idxp_v, buf_v, z_v, acc, sem):
        cid = lax.axis_index("c")
        sid = lax.axis_index("s")

        def zr(i, c):
            z_v[i] = jnp.zeros((D,), jnp.float32)
            return c
        lax.fori_loop(0, 128, zr, 0)
        for p in range(4):
            lo = (cid * 4 + p) * ck
            def zs(i, c):
                acc_r = (sid * (acc_rows // 16 // 128) + i) * 128
                pltpu.sync_copy(z_v, acc.at[pl.ds(acc_r, 128)])
                return c
            lax.fori_loop(0, acc_rows // 16 // 128, zs, 0)
            plsc.subcore_barrier()

            def sweep(i, c):
                ch = i * 16 + sid

                @pl.when(ch < nch)
                def _():
                    pltpu.sync_copy(idx.at[pl.ds(ch * chr_, chr_)], idx_v)
                    pltpu.sync_copy(msg.at[pl.ds(ch * che, che)], buf_v)
                    for j in range(chr_):
                        for g in range(8):
                            iv = idx_v[j, pl.ds(g * 16, 16)]
                            ok = (iv >= lo) & (iv < lo + ck)
                            idxp_v[j, pl.ds(g * 16, 16)] = jnp.where(
                                ok, iv - lo, dump)
                    for j in range(chr_):
                        pltpu.sync_copy(buf_v.at[pl.ds(j * 128, 128)],
                                        acc.at[idxp_v.at[j]], add=True)
                return c
            lax.fori_loop(0, npw, sweep, 0)
            plsc.subcore_barrier()

            def fl(i, c):
                pi = i * 16 + sid

                @pl.when(pi < nfp)
                def _():
                    r = pi * fp
                    pltpu.sync_copy(acc.at[pl.ds(r, fp)],
                                    buf_v.at[pl.ds(0, fp)])
                    pltpu.sync_copy(buf_v.at[pl.ds(0, fp)],
                                    out.at[pl.ds(lo + r, fp)])
                return c
            lax.fori_loop(0, -(-nfp // 16), fl, 0)
            plsc.subcore_barrier()

    return k(msgs, idx2)


def _sc_scatter128(msgs, idx2, n_seg):
    N, D = msgs.shape
    R = idx2.shape[0]
    chr_ = 4
    che = chr_ * 128
    nch = R // chr_
    npw = -(-nch // _NW)
    ck = n_seg // 4
    acc_rows = ck + 444
    dump = ck + 300
    fp = 125
    nfp = ck // fp

    @functools.partial(
        pl.kernel, mesh=_MESH,
        out_type=jax.ShapeDtypeStruct((2, n_seg, D), jnp.float32),
        scratch_types=[
            pltpu.VMEM((chr_, 128), jnp.int32),
            pltpu.VMEM((chr_, 128), jnp.int32),
            pltpu.VMEM((che, D), jnp.float32),
            pltpu.VMEM((128, D), jnp.float32),
            pltpu.VMEM_SHARED((acc_rows, D), jnp.float32),
            pltpu.SemaphoreType.DMA,
        ],
        compiler_params=_NOTILE,
    )
    def k(msg, idx, out, idx_v, idxp_v, buf_v, z_v, acc, sem):
        cid = lax.axis_index("c")
        sid = lax.axis_index("s")

        def zr(i, c):
            z_v[i] = jnp.zeros((D,), jnp.float32)
            return c
        lax.fori_loop(0, 128, zr, 0)
        for p in range(4):
            lo = p * ck

            def zs(i, c):
                pi = i * 16 + sid

                @pl.when(pi < acc_rows // 128)
                def _():
                    pltpu.sync_copy(z_v, acc.at[pl.ds(pi * 128, 128)])
                return c
            lax.fori_loop(0, -(-(acc_rows // 128) // 16), zs, 0)
            plsc.subcore_barrier()

            def sweep(i, c):
                ch = i * _NW + sid * 2 + cid

                @pl.when(ch < nch)
                def _():
                    pltpu.sync_copy(idx.at[pl.ds(ch * chr_, chr_)], idx_v)
                    pltpu.sync_copy(msg.at[pl.ds(ch * che, che)], buf_v)
                    for j in range(chr_):
                        for g in range(8):
                            iv = idx_v[j, pl.ds(g * 16, 16)]
                            ok = (iv >= lo) & (iv < lo + ck)
                            idxp_v[j, pl.ds(g * 16, 16)] = jnp.where(
                                ok, iv - lo, dump)
                    for j in range(chr_):
                        pltpu.sync_copy(buf_v.at[pl.ds(j * 128, 128)],
                                        acc.at[idxp_v.at[j]], add=True)
                return c
            lax.fori_loop(0, npw, sweep, 0)
            plsc.subcore_barrier()

            def fl(i, c):
                pi = i * 16 + sid

                @pl.when(pi < nfp)
                def _():
                    r = pi * fp
                    pltpu.sync_copy(acc.at[pl.ds(r, fp)],
                                    buf_v.at[pl.ds(0, fp)])
                    pltpu.sync_copy(buf_v.at[pl.ds(0, fp)],
                                    out.at[cid].at[pl.ds(lo + r, fp)])
                return c
            lax.fori_loop(0, -(-nfp // 16), fl, 0)
            plsc.subcore_barrier()

    return k(msgs, idx2)


def _rowmm(xs, Ws, b, adds=(), res=None, relu=False, blk=4000):
    n = xs[0].shape[0]
    M = Ws[0].shape[1]
    Wcat = jnp.concatenate(Ws, axis=0)
    K = Wcat.shape[0]
    b8 = jnp.tile(b.reshape(1, M), (8, 1))
    n_x = len(xs)
    n_a = len(adds)
    has_res = res is not None
    grid = n // blk

    def body(*refs):
        xr = refs[:n_x]
        Wr = refs[n_x]
        br = refs[n_x + 1]
        ar = refs[n_x + 2:n_x + 2 + n_a]
        rr = refs[n_x + 2 + n_a] if has_res else None
        out = refs[-1]
        k0 = 0
        acc = None
        for x in xr:
            k = x.shape[1]
            p = jnp.dot(x[...], Wr[pl.ds(k0, k), :],
                        preferred_element_type=jnp.float32)
            acc = p if acc is None else acc + p
            k0 += k
        acc = acc + br[0:1, :]
        for a, col in zip(ar, add_cols):
            acc = acc + a[...][:, col * M:(col + 1) * M]
        if relu:
            acc = jnp.maximum(acc, 0.0)
        if has_res:
            acc = acc + rr[...]
        out[...] = acc

    in_specs = [pl.BlockSpec((blk, x.shape[1]), lambda i: (i, 0)) for x in xs]
    in_specs.append(pl.BlockSpec((K, M), lambda i: (0, 0)))
    in_specs.append(pl.BlockSpec((8, M), lambda i: (0, 0)))
    operands = list(xs) + [Wcat, b8]
    add_cols = []
    for a in adds:
        arr, col = a if isinstance(a, tuple) else (a, 0)
        in_specs.append(pl.BlockSpec((blk, arr.shape[1]), lambda i: (i, 0)))
        operands.append(arr)
        add_cols.append(col)
    if has_res:
        in_specs.append(pl.BlockSpec((blk, M), lambda i: (i, 0)))
        operands.append(res)
    return pl.pallas_call(
        body,
        grid=(grid,),
        in_specs=in_specs,
        out_specs=pl.BlockSpec((blk, M), lambda i: (i, 0)),
        out_shape=jax.ShapeDtypeStruct((n, M), jnp.float32),
    )(*operands)


def _stats16(v, bc, blk=8000):
    n = v.shape[0]
    grid = n // blk

    def body(v_ref, b_ref, out_ref):
        i = pl.program_id(0)
        vv = v_ref[...]
        gid = lax.broadcasted_iota(jnp.int32, (blk, NUM_GRAPHS), 1)
        oh = (b_ref[...] == gid).astype(jnp.float32)
        rs = jnp.sum(vv, axis=1, keepdims=True)
        rq = jnp.sum(vv * vv, axis=1, keepdims=True)
        s = jnp.sum(oh * rs, axis=0, keepdims=True)
        q = jnp.sum(oh * rq, axis=0, keepdims=True)
        c = jnp.sum(oh, axis=0, keepdims=True)
        st = jnp.concatenate(
            [s, q, c, jnp.zeros((5, NUM_GRAPHS), jnp.float32)], axis=0)

        @pl.when(i == 0)
        def _():
            out_ref[...] = st

        @pl.when(i > 0)
        def _():
            out_ref[...] = out_ref[...] + st

    return pl.pallas_call(
        body,
        grid=(grid,),
        in_specs=[pl.BlockSpec((blk, v.shape[1]), lambda i: (i, 0)),
                  pl.BlockSpec((blk, 1), lambda i: (i, 0))],
        out_specs=pl.BlockSpec((8, NUM_GRAPHS), lambda i: (0, 0)),
        out_shape=jax.ShapeDtypeStruct((8, NUM_GRAPHS), jnp.float32),
    )(v, bc)


def _apply_norm(v, bc, st, w, bias, blk=8000):
    n, F = v.shape
    grid = n // blk
    w8 = jnp.tile(w.reshape(1, F), (8, 1))
    b8 = jnp.tile(bias.reshape(1, F), (8, 1))

    def body(v_ref, b_ref, st_ref, w_ref, bb_ref, out_ref):
        s = st_ref[0:1, :]
        q = st_ref[1:2, :]
        c = st_ref[2:3, :]
        denom = jnp.maximum(c * F, 1.0)
        mean = s / denom
        var = q / denom - mean * mean
        inv = lax.rsqrt(var + EPS)
        gid = lax.broadcasted_iota(jnp.int32, (blk, NUM_GRAPHS), 1)
        oh = (b_ref[...] == gid).astype(jnp.float32)
        mean_r = jnp.sum(oh * mean, axis=1, keepdims=True)
        inv_r = jnp.sum(oh * inv, axis=1, keepdims=True)
        out_ref[...] = ((v_ref[...] - mean_r) * inv_r * w_ref[0:1, :]
                        + bb_ref[0:1, :])

    return pl.pallas_call(
        body,
        grid=(grid,),
        in_specs=[pl.BlockSpec((blk, F), lambda i: (i, 0)),
                  pl.BlockSpec((blk, 1), lambda i: (i, 0)),
                  pl.BlockSpec((8, NUM_GRAPHS), lambda i: (0, 0)),
                  pl.BlockSpec((8, F), lambda i: (0, 0)),
                  pl.BlockSpec((8, F), lambda i: (0, 0))],
        out_specs=pl.BlockSpec((blk, F), lambda i: (i, 0)),
        out_shape=jax.ShapeDtypeStruct((n, F), jnp.float32),
    )(v, bc, st, w8, b8)


def _graph_norm(v, batch, w, bias):
    F = v.shape[-1]
    cnt = jax.ops.segment_sum(jnp.ones((v.shape[0],), jnp.float32), batch,
                              num_segments=NUM_GRAPHS)
    denom = jnp.maximum(cnt * F, 1.0)
    mean = jax.ops.segment_sum(v.sum(-1), batch, num_segments=NUM_GRAPHS) / denom
    var = (jax.ops.segment_sum((v * v).sum(-1), batch, num_segments=NUM_GRAPHS)
           / denom - mean * mean)
    inv = 1.0 / jnp.sqrt(var + EPS)
    return (v - mean[batch][:, None]) * inv[batch][:, None] * w + bias


def kernel(x, edge_attr, angle_attr, params, node_batch, edge_index,
           edge_batch, threebody_index, angle_batch):
    D = x.shape[1]
    ED = edge_attr.shape[1]
    N_E = edge_attr.shape[0]
    N_T = angle_attr.shape[0]
    src_e2 = edge_index[0].reshape(N_E // 128, 128)
    dst_e2 = edge_index[1].reshape(N_E // 128, 128)
    src_t2 = threebody_index[0].reshape(N_T // 128, 128)
    dst_t2 = threebody_index[1].reshape(N_T // 128, 128)
    src_e, dst_e = edge_index[0], edge_index[1]
    dst_t = threebody_index[1]
    eb_col = edge_batch.astype(jnp.int32).reshape(-1, 1)
    ab_col = angle_batch.astype(jnp.int32).reshape(-1, 1)
    e = edge_attr
    a = angle_attr
    for lp in params['layers']:
        Wa, Wb, Wc = lp['ne_W'][:D], lp['ne_W'][D:2 * D], lp['ne_W'][2 * D:]
        A, B, C = lp['ea_W'][:ED], lp['ea_W'][ED:2 * ED], lp['ea_W'][2 * ED:]
        Wm1, Wm2 = lp['emp_Wm'][:ED], lp['emp_Wm'][ED:]
        Wu1, Wu2 = lp['emp_Wu'][:ED], lp['emp_Wu'][ED:]
        Nm1, Nm2 = lp['nmp_Wm'][:D], lp['nmp_Wm'][D:]
        Nu1, Nu2 = lp['nmp_Wu'][:D], lp['nmp_Wu'][D:]

        xab = _rowmm([x], [jnp.concatenate([Wa, Wb], axis=1)],
                     jnp.zeros((32,), jnp.float32), blk=1000)
        xm = _rowmm([x], [Nm1], jnp.zeros((128,), jnp.float32), blk=1000)

        ga = _sc_gather(xab[:, :ED], src_e2, 10)
        gb = _sc_gather(xab[:, ED:], dst_e2, 10)
        e_pre = _rowmm([e], [Wc], lp['ne_b'], adds=(ga, gb), res=e)
        e1 = _apply_norm(e_pre, eb_col, _stats16(e_pre, eb_col),
                         lp['en_w'], lp['en_b'])

        epA = _rowmm([e1], [jnp.concatenate([A, Wm1], axis=1)],
                     jnp.zeros((32,), jnp.float32))
        epB = _rowmm([e1], [B], jnp.zeros((ED,), jnp.float32))

        gAM = _sc_gather(epA, src_t2, 10)
        gB = _sc_gather(epB, dst_t2, 10)

        a_pre = _rowmm([a], [C], lp['ea_b'], adds=((gAM, 0), gB), res=a)
        a1 = _apply_norm(a_pre, ab_col, _stats16(a_pre, ab_col),
                         lp['an_w'], lp['an_b'])

        m = _rowmm([a1], [Wm2], lp['emp_bm'], adds=((gAM, 1),), relu=True)
        agg_e = jax.ops.segment_sum(m, dst_t, num_segments=N_E)
        e2 = _rowmm([e1, agg_e], [Wu1, Wu2], lp['emp_bu'], relu=True, res=e1)

        g4 = _sc_gather(xm, src_e2, 4)
        m2 = _rowmm([e2], [Nm2], lp['nmp_bm'], adds=(g4,), relu=True)
        agg_n = jax.ops.segment_sum(m2, dst_e, num_segments=x.shape[0])
        x = _rowmm([x, agg_n], [Nu1, Nu2], lp['nmp_bu'], relu=True, res=x,
                   blk=1000)
        e = e2
        a = a1
    return x
